# baseline (speedup 1.0000x reference)
import functools

import jax
import jax.numpy as jnp
from jax import lax
from jax.experimental import pallas as pl
from jax.experimental.pallas import tpu as pltpu
from jax.experimental.pallas import tpu_sc as plsc

N = 10000
E = 320000
D = 128

NC = 2
NS = 16
NW = NC * NS

C = 50
NCHUNK = E // C
CPT = NCHUNK // NW
NG = 5
GK = CPT // NG
ROWS_A = 624
ROWS_LAST = N - 15 * ROWS_A

_SC_MESH = plsc.VectorSubcoreMesh(core_axis_name="c", subcore_axis_name="s")


@functools.partial(
    pl.kernel,
    out_type=jax.ShapeDtypeStruct((NC, N, D), jnp.float32),
    mesh=_SC_MESH,
    scratch_types=[
        pltpu.VMEM((GK, 1, C), jnp.int32),
        pltpu.VMEM((GK, 1, C), jnp.int32),
        pltpu.VMEM((C, D), jnp.float32),
        pltpu.VMEM((C, D), jnp.float32),
        pltpu.VMEM((C, D), jnp.float32),
        pltpu.VMEM((C, D), jnp.float32),
        pltpu.VMEM_SHARED((N, D), jnp.float32),
        pltpu.SemaphoreType.DMA,
        pltpu.SemaphoreType.DMA,
        pltpu.SemaphoreType.DMA,
        pltpu.SemaphoreType.DMA,
    ],
)
def _sc_agg(x_hbm, zero_hbm, src_hbm, dst_hbm, out_hbm,
            src_v, dst_v, rows0, rows1, rows2, rows3, acc_sh,
            sem0, sem1, sem2, sem3):
    cid = lax.axis_index("c")
    sid = lax.axis_index("s")
    wid = sid * NC + cid

    rows = (rows0, rows1, rows2, rows3)
    sems = (sem0, sem1, sem2, sem3)

    r0 = sid * ROWS_A

    @pl.when(jnp.logical_and(cid == 0, sid < NS - 1))
    def _():
        pltpu.sync_copy(x_hbm.at[pl.ds(r0, ROWS_A)],
                        acc_sh.at[pl.ds(r0, ROWS_A)])

    @pl.when(jnp.logical_and(cid == 0, sid == NS - 1))
    def _():
        pltpu.sync_copy(x_hbm.at[pl.ds(15 * ROWS_A, ROWS_LAST)],
                        acc_sh.at[pl.ds(15 * ROWS_A, ROWS_LAST)])

    @pl.when(jnp.logical_and(cid == 1, sid < NS - 1))
    def _():
        pltpu.sync_copy(zero_hbm.at[pl.ds(r0, ROWS_A)],
                        acc_sh.at[pl.ds(r0, ROWS_A)])

    @pl.when(jnp.logical_and(cid == 1, sid == NS - 1))
    def _():
        pltpu.sync_copy(zero_hbm.at[pl.ds(15 * ROWS_A, ROWS_LAST)],
                        acc_sh.at[pl.ds(15 * ROWS_A, ROWS_LAST)])

    plsc.subcore_barrier()

    def gather_start(j, b):
        pltpu.async_copy(x_hbm.at[src_v.at[j, 0]], rows[b], sems[b])

    def gather_wait(j, b):
        pltpu.make_async_copy(x_hbm.at[src_v.at[j, 0]], rows[b], sems[b]).wait()

    def scatter_add(j, b):
        pltpu.sync_copy(rows[b], acc_sh.at[dst_v.at[j, 0]], add=True)

    for g in range(NG):
        off = CPT * wid + g * GK
        pltpu.sync_copy(src_hbm.at[pl.ds(off, GK)], src_v)
        pltpu.sync_copy(dst_hbm.at[pl.ds(off, GK)], dst_v)

        gather_start(0, 0)
        gather_start(1, 1)
        gather_start(2, 2)

        def body(jj, carry):
            j = 4 * jj
            for u in range(4):
                gather_start(j + u + 3, (u + 3) % 4)
                gather_wait(j + u, u)
                scatter_add(j + u, u)
            return carry

        lax.fori_loop(0, GK // 4 - 1, body, 0)

        gather_start(GK - 1, (GK - 1) % 4)
        for u in range(4):
            gather_wait(GK - 4 + u, (GK - 4 + u) % 4)
            scatter_add(GK - 4 + u, (GK - 4 + u) % 4)

    plsc.subcore_barrier()

    @pl.when(sid < NS - 1)
    def _():
        pltpu.sync_copy(acc_sh.at[pl.ds(r0, ROWS_A)],
                        out_hbm.at[cid, pl.ds(r0, ROWS_A)])

    @pl.when(sid == NS - 1)
    def _():
        pltpu.sync_copy(acc_sh.at[pl.ds(15 * ROWS_A, ROWS_LAST)],
                        out_hbm.at[cid, pl.ds(15 * ROWS_A, ROWS_LAST)])


BLK = 2000


def _tc_mlp_body(final_relu, p_ref, w1_ref, b1_ref, w2_ref, b2_ref, out_ref):
    h = p_ref[0] + p_ref[1]
    t = jnp.dot(h, w1_ref[...], preferred_element_type=jnp.float32) + b1_ref[...]
    t = jnp.maximum(t, 0.0)
    o = jnp.dot(t, w2_ref[...], preferred_element_type=jnp.float32) + b2_ref[...]
    if final_relu:
        o = jnp.maximum(o, 0.0)
    out_ref[...] = o


def _tc_mlp(p, w1, b1, w2, b2, final_relu):
    return pl.pallas_call(
        functools.partial(_tc_mlp_body, final_relu),
        grid=(N // BLK,),
        in_specs=[
            pl.BlockSpec((NC, BLK, D), lambda i: (0, i, 0)),
            pl.BlockSpec((D, D), lambda i: (0, 0)),
            pl.BlockSpec((1, D), lambda i: (0, 0)),
            pl.BlockSpec((D, D), lambda i: (0, 0)),
            pl.BlockSpec((1, D), lambda i: (0, 0)),
        ],
        out_specs=pl.BlockSpec((BLK, D), lambda i: (i, 0)),
        out_shape=jax.ShapeDtypeStruct((N, D), jnp.float32),
    )(p, w1, b1, w2, b2)


def kernel(x, edge_index, W1a, b1a, g1a, be1a, W2a, b2a, gbn0, bbn0,
           W1b, b1b, g1b, be1b, W2b, b2b):
    src = edge_index[0].reshape(NCHUNK, 1, C)
    dst = edge_index[1].reshape(NCHUNK, 1, C)
    zero = jnp.zeros((N, D), jnp.float32)

    c = 1.0 / jnp.sqrt(jnp.float32(1.0 + 1e-5))
    w1a = W1a * (g1a * c)[None, :]
    b1a_f = (b1a * g1a * c + be1a)[None, :]
    w2a = W2a * (gbn0 * c)[None, :]
    b2a_f = (b2a * gbn0 * c + bbn0)[None, :]
    w1b = W1b * (g1b * c)[None, :]
    b1b_f = (b1b * g1b * c + be1b)[None, :]
    b2b_f = b2b[None, :]

    p1 = _sc_agg(x, zero, src, dst)
    h = _tc_mlp(p1, w1a, b1a_f, w2a, b2a_f, final_relu=True)
    p2 = _sc_agg(h, zero, src, dst)
    out = _tc_mlp(p2, w1b, b1b_f, W2b, b2b_f, final_relu=False)
    return out
